# baseline (speedup 1.0000x reference)
import functools

import jax
import jax.numpy as jnp
from jax import lax
from jax.experimental import pallas as pl
from jax.experimental.pallas import tpu as pltpu
from jax.experimental.pallas import tpu_sc as plsc

BATCH = 16384
EMBED_DIM = 32
NC = 2
NS = 16
NW = NC * NS
BPW = BATCH // NW
CHUNK = 128
NCH = BPW // CHUNK


def _sc_body(uidx_hbm, pidx_hbm, uemb_hbm, ub_hbm, pemb_hbm, pb_hbm,
             partials_hbm, bsum_hbm,
             uidx_v, pidx_v, urows_v, prows_v, ubv, pbv, bsumv, accv, sem):
    wid = lax.axis_index("c") * NS + lax.axis_index("s")
    pltpu.sync_copy(uidx_hbm.at[pl.ds(wid * NCH, NCH)], uidx_v)
    pltpu.sync_copy(pidx_hbm.at[pl.ds(wid * NCH, NCH)], pidx_v)
    copies = []
    for j in range(NCH):
        dst = pl.ds(j * CHUNK, CHUNK)
        copies.append(pltpu.async_copy(uemb_hbm.at[uidx_v.at[j]], urows_v.at[dst], sem))
        copies.append(pltpu.async_copy(pemb_hbm.at[pidx_v.at[j]], prows_v.at[dst], sem))
        copies.append(pltpu.async_copy(ub_hbm.at[uidx_v.at[j]], ubv.at[dst], sem))
        copies.append(pltpu.async_copy(pb_hbm.at[pidx_v.at[j]], pbv.at[dst], sem))
    for c in copies:
        c.wait()

    def dot_body(i, acc):
        r = i * 4
        for t in range(4):
            acc = acc + urows_v[r + t, pl.ds(0, 16)] * prows_v[r + t, pl.ds(0, 16)]
            acc = acc + urows_v[r + t, pl.ds(16, 16)] * prows_v[r + t, pl.ds(16, 16)]
        return acc

    acc = lax.fori_loop(0, BPW // 4, dot_body, jnp.zeros((16,), jnp.float32))
    accv[...] = acc

    def bias_body(i, carry):
        s = pl.ds(i * 16, 16)
        bsumv[s] = ubv[s] + pbv[s]
        return carry

    lax.fori_loop(0, BPW // 16, bias_body, 0)
    pltpu.sync_copy(accv, partials_hbm.at[wid])
    pltpu.sync_copy(bsumv, bsum_hbm.at[pl.ds(wid * BPW, BPW)])


def _sc_stage(u_idx, p_idx, user_embedding, ub_flat, place_embedding, pb_flat):
    mesh = plsc.VectorSubcoreMesh(core_axis_name="c", subcore_axis_name="s")
    f = pl.kernel(
        _sc_body,
        mesh=mesh,
        compiler_params=pltpu.CompilerParams(use_tc_tiling_on_sc=False),
        out_type=[
            jax.ShapeDtypeStruct((NW, 16), jnp.float32),
            jax.ShapeDtypeStruct((BATCH,), jnp.float32),
        ],
        scratch_types=[
            pltpu.VMEM((NCH, CHUNK), jnp.int32),
            pltpu.VMEM((NCH, CHUNK), jnp.int32),
            pltpu.VMEM((BPW, EMBED_DIM), jnp.float32),
            pltpu.VMEM((BPW, EMBED_DIM), jnp.float32),
            pltpu.VMEM((BPW,), jnp.float32),
            pltpu.VMEM((BPW,), jnp.float32),
            pltpu.VMEM((BPW,), jnp.float32),
            pltpu.VMEM((16,), jnp.float32),
            pltpu.SemaphoreType.DMA,
        ],
    )
    return f(u_idx, p_idx, user_embedding, ub_flat, place_embedding, pb_flat)


def _tc_body(partials_ref, bsum_ref, out_ref):
    s = jnp.sum(partials_ref[...])
    out_ref[...] = jax.nn.sigmoid(bsum_ref[...] + s)


def kernel(inputs, user_embedding, user_bias, place_embedding, place_bias):
    u_idx = inputs[:, 0].reshape(BATCH // CHUNK, CHUNK)
    p_idx = inputs[:, 1].reshape(BATCH // CHUNK, CHUNK)
    partials, bsum = _sc_stage(
        u_idx, p_idx, user_embedding, user_bias.reshape(-1),
        place_embedding, place_bias.reshape(-1))
    out = pl.pallas_call(
        _tc_body,
        out_shape=jax.ShapeDtypeStruct((128, 128), jnp.float32),
    )(partials, bsum.reshape(128, 128))
    return out.reshape(BATCH, 1)
